# baseline (speedup 1.0000x reference)
import functools

import jax
import jax.numpy as jnp
from jax import lax
from jax.experimental import pallas as pl
from jax.experimental.pallas import tpu as pltpu
from jax.experimental.pallas import tpu_sc as plsc

ROWS = 16384
COLS = 100
EMB = 16
N = ROWS * COLS
NUM_CORES = 2
NUM_SUBCORES = 16
NW = NUM_CORES * NUM_SUBCORES
PER_W = N // NW
CHUNK = 2048
NCHUNK = PER_W // CHUNK
GROUPS = CHUNK // 16
GB = 128
NGB = CHUNK // GB

BINS = (0.0, 0.2, 0.4, 0.6, 0.8, 1.0)
PADDING_IDX = 6

_mesh = plsc.VectorSubcoreMesh(core_axis_name="c", subcore_axis_name="s")


@functools.partial(
    pl.kernel,
    mesh=_mesh,
    out_type=jax.ShapeDtypeStruct((N, EMB), jnp.float32),
    scratch_types=[
        pltpu.VMEM((CHUNK,), jnp.float32),
        pltpu.VMEM((CHUNK,), jnp.int32),
        pltpu.VMEM((CHUNK, EMB), jnp.float32),
        pltpu.SemaphoreType.DMA,
    ],
    compiler_params=pltpu.CompilerParams(
        needs_layout_passes=False, use_tc_tiling_on_sc=False),
)
def _sc_embed(param_hbm, emb_hbm, out_hbm, in_v, idx_v, rows_v, sem):
    wid = lax.axis_index("s") * NUM_CORES + lax.axis_index("c")
    base = wid * PER_W

    ones = jnp.full((16,), 1, jnp.int32)
    zeros = jnp.full((16,), 0, jnp.int32)
    pad_vec = jnp.full((16,), PADDING_IDX, jnp.int32)
    bin_vecs = [jnp.full((16,), b, jnp.float32) for b in BINS]

    def chunk_body(c, carry):
        off = base + c * CHUNK
        pltpu.sync_copy(param_hbm.at[pl.ds(off, CHUNK)], in_v)

        def group_body(g, carry2):
            v = in_v[pl.ds(g * 16, 16)]
            idx = zeros
            for bv in bin_vecs:
                idx = idx + jnp.where(v > bv, ones, zeros)
            idx = jnp.where(v != v, pad_vec, idx)
            idx_v[pl.ds(g * 16, 16)] = idx
            return carry2

        lax.fori_loop(0, GROUPS, group_body, 0)

        copies = [
            pltpu.async_copy(
                emb_hbm.at[idx_v.at[pl.ds(j * GB, GB)]],
                rows_v.at[pl.ds(j * GB, GB), :],
                sem,
            )
            for j in range(NGB)
        ]
        for cp in copies:
            cp.wait()
        pltpu.sync_copy(rows_v, out_hbm.at[pl.ds(off, CHUNK), :])
        return carry

    lax.fori_loop(0, NCHUNK, chunk_body, 0)


def kernel(param, emb_weight):
    out = _sc_embed(param.reshape(-1), emb_weight)
    return out.reshape(ROWS, COLS, EMB)


# speedup vs baseline: 8.9820x; 8.9820x over previous
import functools

import jax
import jax.numpy as jnp
from jax import lax
from jax.experimental import pallas as pl
from jax.experimental.pallas import tpu as pltpu
from jax.experimental.pallas import tpu_sc as plsc

ROWS = 16384
COLS = 100
EMB = 16
N = ROWS * COLS
NUM_CORES = 2
NUM_SUBCORES = 16
NW = NUM_CORES * NUM_SUBCORES
PER_W = N // NW
CHUNK = 2048
NCHUNK = PER_W // CHUNK
GROUPS = CHUNK // 16

BINS = (0.0, 0.2, 0.4, 0.6, 0.8, 1.0)
PADDING_IDX = 6

_mesh = plsc.VectorSubcoreMesh(core_axis_name="c", subcore_axis_name="s")


@functools.partial(
    pl.kernel,
    mesh=_mesh,
    out_type=jax.ShapeDtypeStruct((N * EMB,), jnp.float32),
    scratch_types=[
        pltpu.VMEM((7 * EMB,), jnp.float32),
        pltpu.VMEM((CHUNK,), jnp.float32),
        pltpu.VMEM((CHUNK * EMB,), jnp.float32),
    ],
    compiler_params=pltpu.CompilerParams(
        needs_layout_passes=False, use_tc_tiling_on_sc=False),
)
def _sc_embed(param_hbm, emb_hbm, out_hbm, emb_v, in_v, out_v):
    wid = lax.axis_index("s") * NUM_CORES + lax.axis_index("c")
    base = wid * PER_W
    pltpu.sync_copy(emb_hbm, emb_v)

    iota = lax.iota(jnp.int32, 16)
    ones = jnp.full((16,), 1, jnp.int32)
    zeros = jnp.full((16,), 0, jnp.int32)
    pad_vec = jnp.full((16,), PADDING_IDX, jnp.int32)
    bin_vecs = [jnp.full((16,), b, jnp.float32) for b in BINS]
    sixteen = jnp.full((16,), EMB, jnp.int32)
    row_stride = iota * sixteen
    perms = [iota ^ jnp.full((16,), j, jnp.int32) for j in range(EMB)]

    def chunk_body(c, carry):
        off = base + c * CHUNK
        pltpu.sync_copy(param_hbm.at[pl.ds(off, CHUNK)], in_v)

        def group_body(g, carry2):
            v = in_v[pl.ds(g * 16, 16)]
            idx = zeros
            for bv in bin_vecs:
                idx = idx + jnp.where(v > bv, ones, zeros)
            idx = jnp.where(v != v, pad_vec, idx)
            idx16 = idx * sixteen
            rowbase = row_stride + jnp.full((16,), g * (16 * EMB), jnp.int32)
            for j in range(EMB):
                blk = plsc.load_gather(emb_v, [idx16 + perms[j]])
                plsc.store_scatter(out_v, [rowbase + perms[j]], blk)
            return carry2

        lax.fori_loop(0, GROUPS, group_body, 0)
        pltpu.sync_copy(out_v, out_hbm.at[pl.ds(off * EMB, CHUNK * EMB)])
        return carry

    lax.fori_loop(0, NCHUNK, chunk_body, 0)


def kernel(param, emb_weight):
    out = _sc_embed(param.reshape(-1), emb_weight.reshape(-1))
    return out.reshape(ROWS, COLS, EMB)


# speedup vs baseline: 35.2361x; 3.9230x over previous
import functools

import jax
import jax.numpy as jnp
from jax import lax
from jax.experimental import pallas as pl
from jax.experimental.pallas import tpu as pltpu
from jax.experimental.pallas import tpu_sc as plsc

ROWS = 16384
COLS = 100
EMB = 16
NUM_CORES = 2
NUM_SUBCORES = 16
NW = NUM_CORES * NUM_SUBCORES
IPW = ROWS // NW
IB = 128
NIB = IPW // IB
JQ = 25
NJQ = COLS // JQ

BINS = (0.0, 0.2, 0.4, 0.6, 0.8, 1.0)
PADDING_IDX = 6

_mesh = plsc.VectorSubcoreMesh(core_axis_name="c", subcore_axis_name="s")


@functools.partial(
    pl.kernel,
    mesh=_mesh,
    out_type=jax.ShapeDtypeStruct((COLS, EMB, ROWS), jnp.float32),
    scratch_types=[
        pltpu.VMEM((EMB * 8,), jnp.float32),
        pltpu.VMEM((IB * COLS,), jnp.float32),
        pltpu.VMEM((JQ, 8, IB), jnp.float32),
        pltpu.VMEM((JQ, 8, IB), jnp.float32),
    ],
    compiler_params=pltpu.CompilerParams(needs_layout_passes=False),
)
def _sc_embed(param_hbm, tblt_hbm, out_hbm, tblt_v, in_v, stg_a, stg_b):
    wid = lax.axis_index("s") * NUM_CORES + lax.axis_index("c")
    ibase = wid * IPW
    pltpu.sync_copy(tblt_hbm, tblt_v)

    iota = lax.iota(jnp.int32, 16)
    iota100 = iota * jnp.full((16,), COLS, jnp.int32)
    ones = jnp.full((16,), 1, jnp.int32)
    zeros = jnp.full((16,), 0, jnp.int32)
    pad_vec = jnp.full((16,), PADDING_IDX, jnp.int32)
    bin_vecs = [jnp.full((16,), b, jnp.float32) for b in BINS]
    kbase = [jnp.full((16,), k * 8, jnp.int32) for k in range(EMB)]

    def block_body(b, carry):
        i0 = ibase + b * IB
        pltpu.sync_copy(param_hbm.at[pl.ds(i0 * COLS, IB * COLS)], in_v)

        def quarter_body(q, carry2):
            j0 = q * JQ

            def jj_body(jj, carry3):
                j = j0 + jj

                def i16_body(g, carry4):
                    paddr = iota100 + jnp.full((16,), g * (16 * COLS),
                                               jnp.int32) \
                        + jnp.full((16,), j, jnp.int32)
                    v = plsc.load_gather(in_v, [paddr])
                    idx = zeros
                    for bv in bin_vecs:
                        idx = idx + jnp.where(v > bv, ones, zeros)
                    idx = jnp.where(v != v, pad_vec, idx)
                    for k in range(EMB):
                        val = plsc.load_gather(tblt_v, [kbase[k] + idx])
                        tgt = stg_a if k < 8 else stg_b
                        tgt[jj, k % 8, pl.ds(g * 16, 16)] = val
                    return carry4

                lax.fori_loop(0, IB // 16, i16_body, 0)
                return carry3

            lax.fori_loop(0, JQ, jj_body, 0)
            pltpu.sync_copy(
                stg_a, out_hbm.at[pl.ds(j0, JQ), pl.ds(0, 8), pl.ds(i0, IB)])
            pltpu.sync_copy(
                stg_b, out_hbm.at[pl.ds(j0, JQ), pl.ds(8, 8), pl.ds(i0, IB)])
            return carry2

        lax.fori_loop(0, NJQ, quarter_body, 0)
        return carry

    lax.fori_loop(0, NIB, block_body, 0)


def kernel(param, emb_weight):
    tblt = jnp.pad(emb_weight.T, ((0, 0), (0, 1))).reshape(-1)
    out = _sc_embed(param.reshape(-1), tblt)
    return out.transpose(2, 0, 1)


# speedup vs baseline: 40.1141x; 1.1384x over previous
import functools

import jax
import jax.numpy as jnp
from jax import lax
from jax.experimental import pallas as pl
from jax.experimental.pallas import tpu as pltpu
from jax.experimental.pallas import tpu_sc as plsc

ROWS = 16384
COLS = 100
EMB = 16
NUM_CORES = 2
NUM_SUBCORES = 16
NW = NUM_CORES * NUM_SUBCORES
IPW = ROWS // NW
IB = 128
NIB = IPW // IB
JQ = 25
NJQ = COLS // JQ

BINS = (0.0, 0.2, 0.4, 0.6, 0.8, 1.0)
PADDING_IDX = 6

_mesh = plsc.VectorSubcoreMesh(core_axis_name="c", subcore_axis_name="s")


@functools.partial(
    pl.kernel,
    mesh=_mesh,
    out_type=jax.ShapeDtypeStruct((COLS, EMB, ROWS), jnp.float32),
    scratch_types=[
        pltpu.VMEM((EMB * 8,), jnp.float32),
        pltpu.VMEM((IB * COLS,), jnp.float32),
        pltpu.VMEM((JQ, 8, IB), jnp.float32),
        pltpu.VMEM((JQ, 8, IB), jnp.float32),
        pltpu.VMEM((JQ, 8, IB), jnp.float32),
        pltpu.VMEM((JQ, 8, IB), jnp.float32),
        pltpu.SemaphoreType.DMA,
        pltpu.SemaphoreType.DMA,
    ],
    compiler_params=pltpu.CompilerParams(needs_layout_passes=False),
)
def _sc_embed(param_hbm, tblt_hbm, out_hbm, tblt_v, in_v,
              stg_a0, stg_b0, stg_a1, stg_b1, sem0, sem1):
    wid = lax.axis_index("s") * NUM_CORES + lax.axis_index("c")
    ibase = wid * IPW
    pltpu.sync_copy(tblt_hbm, tblt_v)

    iota = lax.iota(jnp.int32, 16)
    iota100 = iota * jnp.full((16,), COLS, jnp.int32)
    ones = jnp.full((16,), 1, jnp.int32)
    zeros = jnp.full((16,), 0, jnp.int32)
    pad_vec = jnp.full((16,), PADDING_IDX, jnp.int32)
    bin_vecs = [jnp.full((16,), b, jnp.float32) for b in BINS]
    kbase = [jnp.full((16,), k * 8, jnp.int32) for k in range(EMB)]

    stgs = [(stg_a0, stg_b0), (stg_a1, stg_b1)]
    sems = [sem0, sem1]
    pending = [None, None]
    t = 0
    for b in range(NIB):
        i0 = ibase + b * IB
        pltpu.sync_copy(param_hbm.at[pl.ds(i0 * COLS, IB * COLS)], in_v)
        for q in range(NJQ):
            j0 = q * JQ
            p = t % 2
            stg_a, stg_b = stgs[p]
            if pending[p] is not None:
                pending[p][0].wait()
                pending[p][1].wait()

            def jj_body(jj, carry3, stg_a=stg_a, stg_b=stg_b, j0=j0):
                j = j0 + jj

                def i16_body(g, carry4):
                    paddr = iota100 + jnp.full((16,), g * (16 * COLS),
                                               jnp.int32) \
                        + jnp.full((16,), j, jnp.int32)
                    v = plsc.load_gather(in_v, [paddr])
                    idx = zeros
                    for bv in bin_vecs:
                        idx = idx + jnp.where(v > bv, ones, zeros)
                    idx = jnp.where(v != v, pad_vec, idx)
                    for k in range(EMB):
                        val = plsc.load_gather(tblt_v, [kbase[k] + idx])
                        tgt = stg_a if k < 8 else stg_b
                        tgt[jj, k % 8, pl.ds(g * 16, 16)] = val
                    return carry4

                lax.fori_loop(0, IB // 16, i16_body, 0, unroll=2)
                return carry3

            lax.fori_loop(0, JQ, jj_body, 0)
            h1 = pltpu.async_copy(
                stg_a, out_hbm.at[pl.ds(j0, JQ), pl.ds(0, 8), pl.ds(i0, IB)],
                sems[p])
            h2 = pltpu.async_copy(
                stg_b, out_hbm.at[pl.ds(j0, JQ), pl.ds(8, 8), pl.ds(i0, IB)],
                sems[p])
            pending[p] = (h1, h2)
            t += 1
    for p in (0, 1):
        if pending[p] is not None:
            pending[p][0].wait()
            pending[p][1].wait()


def kernel(param, emb_weight):
    tblt = jnp.pad(emb_weight.T, ((0, 0), (0, 1))).reshape(-1)
    out = _sc_embed(param.reshape(-1), tblt)
    return out.transpose(2, 0, 1)


# speedup vs baseline: 72.9320x; 1.8181x over previous
import functools

import jax
import jax.numpy as jnp
from jax import lax
from jax.experimental import pallas as pl
from jax.experimental.pallas import tpu as pltpu
from jax.experimental.pallas import tpu_sc as plsc

ROWS = 16384
COLS = 100
EMB = 16
NUM_CORES = 2
NUM_SUBCORES = 16
NW = NUM_CORES * NUM_SUBCORES
IPW = ROWS // NW
IB = 128
NIB = IPW // IB
JQ = 25
NJQ = COLS // JQ

BINS = (0.0, 0.2, 0.4, 0.6, 0.8, 1.0)
PADDING_IDX = 6

_mesh = plsc.VectorSubcoreMesh(core_axis_name="c", subcore_axis_name="s")


@functools.partial(
    pl.kernel,
    mesh=_mesh,
    out_type=jax.ShapeDtypeStruct((COLS, EMB, ROWS), jnp.float32),
    scratch_types=[
        pltpu.VMEM((EMB * 8,), jnp.float32),
        pltpu.VMEM((IB * COLS,), jnp.float32),
        pltpu.VMEM((JQ, 8, IB), jnp.float32),
        pltpu.VMEM((JQ, 8, IB), jnp.float32),
        pltpu.VMEM((JQ, 8, IB), jnp.float32),
        pltpu.VMEM((JQ, 8, IB), jnp.float32),
        pltpu.SemaphoreType.DMA,
        pltpu.SemaphoreType.DMA,
    ],
    compiler_params=pltpu.CompilerParams(needs_layout_passes=False),
)
def _sc_embed(param_hbm, tblt_hbm, out_hbm, tblt_v, in_v,
              stg_a0, stg_b0, stg_a1, stg_b1, sem0, sem1):
    wid = lax.axis_index("s") * NUM_CORES + lax.axis_index("c")
    ibase = wid * IPW
    pltpu.sync_copy(tblt_hbm, tblt_v)

    iota = lax.iota(jnp.int32, 16)
    iota100 = iota * jnp.full((16,), COLS, jnp.int32)
    ones = jnp.full((16,), 1, jnp.int32)
    zeros = jnp.full((16,), 0, jnp.int32)
    pad_vec = jnp.full((16,), PADDING_IDX, jnp.int32)
    bin_vecs = [jnp.full((16,), b, jnp.float32) for b in BINS]
    kbase = [jnp.full((16,), k * 8, jnp.int32) for k in range(EMB)]

    stgs = [(stg_a0, stg_b0), (stg_a1, stg_b1)]
    sems = [sem0, sem1]
    pending = [None, None]
    t = 0
    for b in range(NIB):
        i0 = ibase + b * IB
        pltpu.sync_copy(param_hbm.at[pl.ds(i0 * COLS, IB * COLS)], in_v)
        for q in range(NJQ):
            j0 = q * JQ
            p = t % 2
            stg_a, stg_b = stgs[p]
            if pending[p] is not None:
                pending[p][0].wait()
                pending[p][1].wait()

            def jj_body(jj, carry3, stg_a=stg_a, stg_b=stg_b, j0=j0):
                j = j0 + jj

                def i16_body(g, carry4):
                    paddr = iota100 + jnp.full((16,), g * (16 * COLS),
                                               jnp.int32) \
                        + jnp.full((16,), j, jnp.int32)
                    v = plsc.load_gather(in_v, [paddr])
                    idx = zeros
                    for bv in bin_vecs:
                        idx = idx + jnp.where(v > bv, ones, zeros)
                    idx = jnp.where(v != v, pad_vec, idx)
                    vals = [plsc.load_gather(tblt_v, [kbase[k] + idx])
                            for k in range(EMB)]
                    for k in range(EMB):
                        tgt = stg_a if k < 8 else stg_b
                        tgt[jj, k % 8, pl.ds(g * 16, 16)] = vals[k]
                    return carry4

                lax.fori_loop(0, IB // 16, i16_body, 0, unroll=2)
                return carry3

            lax.fori_loop(0, JQ, jj_body, 0)
            h1 = pltpu.async_copy(
                stg_a, out_hbm.at[pl.ds(j0, JQ), pl.ds(0, 8), pl.ds(i0, IB)],
                sems[p])
            h2 = pltpu.async_copy(
                stg_b, out_hbm.at[pl.ds(j0, JQ), pl.ds(8, 8), pl.ds(i0, IB)],
                sems[p])
            pending[p] = (h1, h2)
            t += 1
    for p in (0, 1):
        if pending[p] is not None:
            pending[p][0].wait()
            pending[p][1].wait()


def kernel(param, emb_weight):
    tblt = jnp.pad(emb_weight.T, ((0, 0), (0, 1))).reshape(-1)
    out = _sc_embed(param.reshape(-1), tblt)
    return out.transpose(2, 0, 1)


# speedup vs baseline: 162.5684x; 2.2290x over previous
import functools

import jax
import jax.numpy as jnp
from jax import lax
from jax.experimental import pallas as pl
from jax.experimental.pallas import tpu as pltpu
from jax.experimental.pallas import tpu_sc as plsc

ROWS = 16384
COLS = 100
EMB = 16
NUM_CORES = 2
NUM_SUBCORES = 16
NW = NUM_CORES * NUM_SUBCORES
IPW = ROWS // NW
IB = 128
NIB = IPW // IB
JQ = 25
NJQ = COLS // JQ

BINS = (0.0, 0.2, 0.4, 0.6, 0.8, 1.0)
PADDING_IDX = 6

_mesh = plsc.VectorSubcoreMesh(core_axis_name="c", subcore_axis_name="s")


@functools.partial(
    pl.kernel,
    mesh=_mesh,
    out_type=jax.ShapeDtypeStruct((COLS, EMB, ROWS), jnp.float32),
    scratch_types=[
        pltpu.VMEM((EMB * 8,), jnp.float32),
        pltpu.VMEM((COLS, IB), jnp.float32),
        pltpu.VMEM((COLS, IB), jnp.float32),
        pltpu.VMEM((JQ, 8, IB), jnp.float32),
        pltpu.VMEM((JQ, 8, IB), jnp.float32),
        pltpu.VMEM((JQ, 8, IB), jnp.float32),
        pltpu.VMEM((JQ, 8, IB), jnp.float32),
        pltpu.SemaphoreType.DMA,
        pltpu.SemaphoreType.DMA,
        pltpu.SemaphoreType.DMA,
    ],
    compiler_params=pltpu.CompilerParams(needs_layout_passes=False),
)
def _sc_embed(param_hbm, tblt_hbm, out_hbm, tblt_v, in_v0, in_v1,
              stg_a0, stg_b0, stg_a1, stg_b1, sem0, sem1, sem_in):
    wid = lax.axis_index("s") * NUM_CORES + lax.axis_index("c")
    ibase = wid * IPW
    pltpu.sync_copy(tblt_hbm, tblt_v)

    ones = jnp.full((16,), 1, jnp.int32)
    zeros = jnp.full((16,), 0, jnp.int32)
    pad_vec = jnp.full((16,), PADDING_IDX, jnp.int32)
    bin_vecs = [jnp.full((16,), b, jnp.float32) for b in BINS]
    kbase = [jnp.full((16,), k * 8, jnp.int32) for k in range(EMB)]

    stgs = [(stg_a0, stg_b0), (stg_a1, stg_b1)]
    sems = [sem0, sem1]
    pending = [None, None]
    in_bufs = [in_v0, in_v1]
    h_in = pltpu.async_copy(param_hbm.at[:, pl.ds(ibase, IB)], in_v0, sem_in)
    t = 0
    for b in range(NIB):
        i0 = ibase + b * IB
        in_v = in_bufs[b % 2]
        h_in.wait()
        if b + 1 < NIB:
            h_in = pltpu.async_copy(
                param_hbm.at[:, pl.ds(i0 + IB, IB)], in_bufs[(b + 1) % 2],
                sem_in)
        for q in range(NJQ):
            j0 = q * JQ
            p = t % 2
            stg_a, stg_b = stgs[p]
            if pending[p] is not None:
                pending[p][0].wait()
                pending[p][1].wait()

            @plsc.parallel_loop(0, JQ)
            def jj_body(jj, stg_a=stg_a, stg_b=stg_b, j0=j0, in_v=in_v):
                j = j0 + jj

                @plsc.parallel_loop(0, IB // 16, unroll=2)
                def i16_body(g):
                    v = in_v[j, pl.ds(g * 16, 16)]
                    idx = zeros
                    for bv in bin_vecs:
                        idx = idx + jnp.where(v > bv, ones, zeros)
                    idx = jnp.where(v != v, pad_vec, idx)
                    vals = [plsc.load_gather(tblt_v, [kbase[k] + idx])
                            for k in range(EMB)]
                    for k in range(EMB):
                        tgt = stg_a if k < 8 else stg_b
                        tgt[jj, k % 8, pl.ds(g * 16, 16)] = vals[k]

            h1 = pltpu.async_copy(
                stg_a, out_hbm.at[pl.ds(j0, JQ), pl.ds(0, 8), pl.ds(i0, IB)],
                sems[p])
            h2 = pltpu.async_copy(
                stg_b, out_hbm.at[pl.ds(j0, JQ), pl.ds(8, 8), pl.ds(i0, IB)],
                sems[p])
            pending[p] = (h1, h2)
            t += 1
    for p in (0, 1):
        if pending[p] is not None:
            pending[p][0].wait()
            pending[p][1].wait()


def kernel(param, emb_weight):
    tblt = jnp.pad(emb_weight.T, ((0, 0), (0, 1))).reshape(-1)
    out = _sc_embed(param.T, tblt)
    return out.transpose(2, 0, 1)
